# baseline (speedup 1.0000x reference)
import functools

import jax
import jax.numpy as jnp
from jax import lax
from jax.experimental import pallas as pl
from jax.experimental.pallas import tpu as pltpu
from jax.experimental.pallas import tpu_sc as plsc

UNITS = 4096
INPUT_DIM = 4096
BATCH = 128
CONN = 167772

NC = 2
NS = 16
LANES = 16
G = 128
NCH = 42
CONN_PER_TEC = NCH * G
C_PAD = NC * NS * CONN_PER_TEC
ROWS_PER_TEC = UNITS // NS

_GATHER_DN = lax.GatherDimensionNumbers(
    offset_dims=(), collapsed_slice_dims=(0,), start_index_map=(0,)
)


def _vbroadcast(vec, k):
  idx = jnp.full((LANES, 1), k, jnp.int32)
  return lax.gather(
      vec, idx, _GATHER_DN, slice_sizes=(1,),
      mode=lax.GatherScatterMode.PROMISE_IN_BOUNDS,
  )


def _sc_spmm(xT, w_p, rows_p, cols_p):
  mesh = plsc.VectorSubcoreMesh(
      core_axis_name="c", subcore_axis_name="s", num_cores=NC, num_subcores=NS
  )

  @functools.partial(
      pl.kernel,
      mesh=mesh,
      out_type=jax.ShapeDtypeStruct((NC * UNITS, BATCH), jnp.float32),
      scratch_types=[
          pltpu.VMEM((G,), jnp.int32),
          pltpu.VMEM((G,), jnp.int32),
          pltpu.VMEM((G,), jnp.int32),
          pltpu.VMEM((G,), jnp.int32),
          pltpu.VMEM((G,), jnp.int32),
          pltpu.VMEM((G,), jnp.int32),
          pltpu.VMEM((G,), jnp.float32),
          pltpu.VMEM((G,), jnp.float32),
          pltpu.VMEM((G,), jnp.float32),
          pltpu.VMEM((G, BATCH), jnp.float32),
          pltpu.VMEM((G, BATCH), jnp.float32),
          pltpu.VMEM((G, BATCH), jnp.float32),
          pltpu.VMEM_SHARED((UNITS, BATCH), jnp.float32),
          pltpu.SemaphoreType.DMA,
          pltpu.SemaphoreType.DMA,
          pltpu.SemaphoreType.DMA,
          pltpu.SemaphoreType.DMA,
          pltpu.SemaphoreType.DMA,
          pltpu.SemaphoreType.DMA,
          pltpu.SemaphoreType.DMA,
          pltpu.SemaphoreType.DMA,
          pltpu.SemaphoreType.DMA,
          pltpu.SemaphoreType.DMA,
          pltpu.SemaphoreType.DMA,
          pltpu.SemaphoreType.DMA,
      ],
  )
  def k(xT_hbm, w_hbm, rows_hbm, cols_hbm, out_hbm,
        rb0, rb1, rb2, cb0, cb1, cb2, wb0, wb1, wb2, xb0, xb1, xb2,
        acc,
        semr0, semr1, semr2, semcw0, semcw1, semcw2,
        semg0, semg1, semg2, sems0, sems1, sems2):
    cid = lax.axis_index("c")
    sid = lax.axis_index("s")
    rb = (rb0, rb1, rb2)
    cb = (cb0, cb1, cb2)
    wb = (wb0, wb1, wb2)
    xb = (xb0, xb1, xb2)
    semr = (semr0, semr1, semr2)
    semcw = (semcw0, semcw1, semcw2)
    semg = (semg0, semg1, semg2)
    sems = (sems0, sems1, sems2)

    base = (cid * NS + sid) * CONN_PER_TEC

    def rows_start(i, b):
      pltpu.async_copy(rows_hbm.at[pl.ds(base + i * G, G)], rb[b], semr[b])

    def rows_wait(i, b):
      pltpu.make_async_copy(
          rows_hbm.at[pl.ds(base + i * G, G)], rb[b], semr[b]
      ).wait()

    def cw_start(i, b):
      pltpu.async_copy(cols_hbm.at[pl.ds(base + i * G, G)], cb[b], semcw[b])
      pltpu.async_copy(w_hbm.at[pl.ds(base + i * G, G)], wb[b], semcw[b])

    def cw_wait(i, b):
      pltpu.make_async_copy(
          cols_hbm.at[pl.ds(base + i * G, G)], cb[b], semcw[b]
      ).wait()
      pltpu.make_async_copy(
          w_hbm.at[pl.ds(base + i * G, G)], wb[b], semcw[b]
      ).wait()

    def gather_start(i, b):
      pltpu.async_copy(xT_hbm.at[rb[b]], xb[b], semg[b])

    def gather_wait(i, b):
      pltpu.make_async_copy(xT_hbm.at[rb[b]], xb[b], semg[b]).wait()

    def scatter_start(i, b):
      pltpu.async_copy(xb[b], acc.at[cb[b]], sems[b], add=True)

    def scatter_wait(i, b):
      pltpu.make_async_copy(xb[b], acc.at[cb[b]], sems[b]).wait()

    rows_start(0, 0)
    cw_start(0, 0)

    zero16 = jnp.zeros((LANES,), jnp.float32)

    def zrow(i, carry):
      for t in range(BATCH // LANES):
        xb1[i, pl.ds(t * LANES, LANES)] = zero16
      return carry

    lax.fori_loop(0, G, zrow, 0)
    rows_wait(0, 0)
    gather_start(0, 0)
    rows_start(1, 1)
    for q in range(ROWS_PER_TEC // G):
      pltpu.sync_copy(
          xb1, acc.at[pl.ds(sid * ROWS_PER_TEC + q * G, G)]
      )
    plsc.subcore_barrier()

    def chunk(i, m):
      m1 = (m + 1) % 3
      m2 = (m + 2) % 3
      with jax.named_scope("gwait"):
        gather_wait(i, m)

      with jax.named_scope("starts"):
        @pl.when(i >= 2)
        def _():
          scatter_wait(i - 2, m1)

        @pl.when(i + 1 < NCH)
        def _():
          rows_wait(i + 1, m1)
          gather_start(i + 1, m1)
          cw_start(i + 1, m1)

        @pl.when(i + 2 < NCH)
        def _():
          rows_start(i + 2, m2)

        cw_wait(i, m)
      xbuf = xb[m]
      wbuf = wb[m]

      with jax.named_scope("scale"):
        for g in range(G // LANES):
          j0 = g * LANES
          wrow = wbuf[pl.ds(j0, LANES)]
          for kk in range(LANES):
            row = j0 + kk
            wk = _vbroadcast(wrow, kk)
            for t in range(BATCH // LANES):
              sl = pl.ds(t * LANES, LANES)
              xbuf[row, sl] = xbuf[row, sl] * wk
      with jax.named_scope("scatter"):
        scatter_start(i, m)

    def triple(i3, carry):
      for u in range(3):
        chunk(i3 * 3 + u, u)
      return carry

    lax.fori_loop(0, NCH // 3, triple, 0)
    scatter_wait(NCH - 2, (NCH - 2) % 3)
    scatter_wait(NCH - 1, (NCH - 1) % 3)

    plsc.subcore_barrier()
    pltpu.sync_copy(
        acc.at[pl.ds(sid * ROWS_PER_TEC, ROWS_PER_TEC)],
        out_hbm.at[pl.ds(cid * UNITS + sid * ROWS_PER_TEC, ROWS_PER_TEC)],
    )

  return k(xT, w_p, rows_p, cols_p)


_UBLK = 512


def _combine_body(eye_ref, p_ref, b_ref, o_ref):
  p = p_ref[0].astype(jnp.float32) + p_ref[1].astype(jnp.float32)
  t = lax.dot_general(
      eye_ref[...], p, (((1,), (1,)), ((), ())),
      preferred_element_type=jnp.float32,
  )
  o_ref[...] = t + b_ref[...]


def _combine(partials, bias):
  eye = jnp.eye(BATCH, dtype=jnp.float32)
  return pl.pallas_call(
      _combine_body,
      grid=(UNITS // _UBLK,),
      in_specs=[
          pl.BlockSpec((BATCH, BATCH), lambda i: (0, 0)),
          pl.BlockSpec((NC, _UBLK, BATCH), lambda i: (0, i, 0)),
          pl.BlockSpec((1, _UBLK), lambda i: (0, i)),
      ],
      out_specs=pl.BlockSpec((BATCH, _UBLK), lambda i: (0, i)),
      out_shape=jax.ShapeDtypeStruct((BATCH, UNITS), jnp.float32),
  )(eye, partials, bias.reshape(1, UNITS))


def kernel(inputs, kernel_weights, rows, cols, bias):
  pad = C_PAD - CONN
  pad_idx = jnp.arange(CONN, C_PAD, dtype=jnp.int32)
  xT = inputs.T
  w_p = jnp.pad(kernel_weights, (0, pad))
  rows_p = jnp.concatenate([rows, pad_idx % INPUT_DIM])
  cols_p = jnp.concatenate([cols, pad_idx % UNITS])
  partials = _sc_spmm(xT, w_p, rows_p, cols_p)
  return _combine(partials.reshape(NC, UNITS, BATCH), bias)


# speedup vs baseline: 1.2425x; 1.2425x over previous
import functools

import jax
import jax.numpy as jnp
from jax import lax
from jax.experimental import pallas as pl
from jax.experimental.pallas import tpu as pltpu
from jax.experimental.pallas import tpu_sc as plsc

UNITS = 4096
INPUT_DIM = 4096
BATCH = 128
CONN = 167772

NC = 2
NS = 16
LANES = 16
G = 128
NCH = 42
CONN_PER_TEC = NCH * G
C_PAD = NC * NS * CONN_PER_TEC
ROWS_PER_TEC = UNITS // NS

_GATHER_DN = lax.GatherDimensionNumbers(
    offset_dims=(), collapsed_slice_dims=(0,), start_index_map=(0,)
)


def _vbroadcast(vec, k):
  idx = jnp.full((LANES, 1), k, jnp.int32)
  return lax.gather(
      vec, idx, _GATHER_DN, slice_sizes=(1,),
      mode=lax.GatherScatterMode.PROMISE_IN_BOUNDS,
  )


def _sc_spmm(xT, w_p, rows_p, cols_p):
  mesh = plsc.VectorSubcoreMesh(
      core_axis_name="c", subcore_axis_name="s", num_cores=NC, num_subcores=NS
  )

  @functools.partial(
      pl.kernel,
      mesh=mesh,
      out_type=jax.ShapeDtypeStruct((NC * UNITS, BATCH), jnp.float32),
      scratch_types=[
          pltpu.VMEM((G,), jnp.int32),
          pltpu.VMEM((G,), jnp.int32),
          pltpu.VMEM((G,), jnp.int32),
          pltpu.VMEM((G,), jnp.int32),
          pltpu.VMEM((G,), jnp.int32),
          pltpu.VMEM((G,), jnp.int32),
          pltpu.VMEM((G,), jnp.float32),
          pltpu.VMEM((G,), jnp.float32),
          pltpu.VMEM((G,), jnp.float32),
          pltpu.VMEM((G, BATCH), jnp.float32),
          pltpu.VMEM((G, BATCH), jnp.float32),
          pltpu.VMEM((G, BATCH), jnp.float32),
          pltpu.VMEM_SHARED((UNITS, BATCH), jnp.float32),
          pltpu.SemaphoreType.DMA,
          pltpu.SemaphoreType.DMA,
          pltpu.SemaphoreType.DMA,
          pltpu.SemaphoreType.DMA,
          pltpu.SemaphoreType.DMA,
          pltpu.SemaphoreType.DMA,
          pltpu.SemaphoreType.DMA,
          pltpu.SemaphoreType.DMA,
          pltpu.SemaphoreType.DMA,
          pltpu.SemaphoreType.DMA,
          pltpu.SemaphoreType.DMA,
          pltpu.SemaphoreType.DMA,
      ],
  )
  def k(xT_hbm, w_hbm, rows_hbm, cols_hbm, out_hbm,
        rb0, rb1, rb2, cb0, cb1, cb2, wb0, wb1, wb2, xb0, xb1, xb2,
        acc,
        semr0, semr1, semr2, semcw0, semcw1, semcw2,
        semg0, semg1, semg2, sems0, sems1, sems2):
    cid = lax.axis_index("c")
    sid = lax.axis_index("s")
    rb = (rb0, rb1, rb2)
    cb = (cb0, cb1, cb2)
    wb = (wb0, wb1, wb2)
    xb = (xb0, xb1, xb2)
    semr = (semr0, semr1, semr2)
    semcw = (semcw0, semcw1, semcw2)
    semg = (semg0, semg1, semg2)
    sems = (sems0, sems1, sems2)

    base = (cid * NS + sid) * CONN_PER_TEC

    def rows_start(i, b):
      pltpu.async_copy(rows_hbm.at[pl.ds(base + i * G, G)], rb[b], semr[b])

    def rows_wait(i, b):
      pltpu.make_async_copy(
          rows_hbm.at[pl.ds(base + i * G, G)], rb[b], semr[b]
      ).wait()

    def cw_start(i, b):
      pltpu.async_copy(cols_hbm.at[pl.ds(base + i * G, G)], cb[b], semcw[b])
      pltpu.async_copy(w_hbm.at[pl.ds(base + i * G, G)], wb[b], semcw[b])

    def cw_wait(i, b):
      pltpu.make_async_copy(
          cols_hbm.at[pl.ds(base + i * G, G)], cb[b], semcw[b]
      ).wait()
      pltpu.make_async_copy(
          w_hbm.at[pl.ds(base + i * G, G)], wb[b], semcw[b]
      ).wait()

    def gather_start(i, b):
      pltpu.async_copy(xT_hbm.at[rb[b]], xb[b], semg[b])

    def gather_wait(i, b):
      pltpu.make_async_copy(xT_hbm.at[rb[b]], xb[b], semg[b]).wait()

    def scatter_start(i, b):
      pltpu.async_copy(xb[b], acc.at[cb[b]], sems[b], add=True)

    def scatter_wait(i, b):
      pltpu.make_async_copy(xb[b], acc.at[cb[b]], sems[b]).wait()

    rows_start(0, 0)
    cw_start(0, 0)

    zero16 = jnp.zeros((LANES,), jnp.float32)

    def zrow(i, carry):
      for t in range(BATCH // LANES):
        xb1[i, pl.ds(t * LANES, LANES)] = zero16
      return carry

    lax.fori_loop(0, G, zrow, 0)
    rows_wait(0, 0)
    gather_start(0, 0)
    rows_start(1, 1)
    for q in range(ROWS_PER_TEC // G):
      pltpu.sync_copy(
          xb1, acc.at[pl.ds(sid * ROWS_PER_TEC + q * G, G)]
      )
    plsc.subcore_barrier()

    def chunk(i, m):
      m1 = (m + 1) % 3
      m2 = (m + 2) % 3
      with jax.named_scope("gwait"):
        gather_wait(i, m)

      with jax.named_scope("starts"):
        @pl.when(i >= 2)
        def _():
          scatter_wait(i - 2, m1)

        @pl.when(i + 1 < NCH)
        def _():
          rows_wait(i + 1, m1)
          gather_start(i + 1, m1)
          cw_start(i + 1, m1)

        @pl.when(i + 2 < NCH)
        def _():
          rows_start(i + 2, m2)

        cw_wait(i, m)
      xbuf = xb[m]
      wbuf = wb[m]

      def scale(g, c2):
        j0 = g * LANES
        wrow = wbuf[pl.ds(j0, LANES)]
        for kk in range(LANES):
          row = j0 + kk
          wk = _vbroadcast(wrow, kk)
          for t in range(BATCH // LANES):
            sl = pl.ds(t * LANES, LANES)
            xbuf[row, sl] = xbuf[row, sl] * wk
        return c2

      with jax.named_scope("scale"):
        lax.fori_loop(0, G // LANES, scale, 0)
      with jax.named_scope("scatter"):
        scatter_start(i, m)

    def triple(i3, carry):
      for u in range(3):
        chunk(i3 * 3 + u, u)
      return carry

    lax.fori_loop(0, NCH // 3, triple, 0)
    scatter_wait(NCH - 2, (NCH - 2) % 3)
    scatter_wait(NCH - 1, (NCH - 1) % 3)

    plsc.subcore_barrier()
    pltpu.sync_copy(
        acc.at[pl.ds(sid * ROWS_PER_TEC, ROWS_PER_TEC)],
        out_hbm.at[pl.ds(cid * UNITS + sid * ROWS_PER_TEC, ROWS_PER_TEC)],
    )

  return k(xT, w_p, rows_p, cols_p)


_UBLK = 512


def _combine_body(eye_ref, p_ref, b_ref, o_ref):
  p = p_ref[0].astype(jnp.float32) + p_ref[1].astype(jnp.float32)
  t = lax.dot_general(
      eye_ref[...], p, (((1,), (1,)), ((), ())),
      preferred_element_type=jnp.float32,
  )
  o_ref[...] = t + b_ref[...]


def _combine(partials, bias):
  eye = jnp.eye(BATCH, dtype=jnp.float32)
  return pl.pallas_call(
      _combine_body,
      grid=(UNITS // _UBLK,),
      in_specs=[
          pl.BlockSpec((BATCH, BATCH), lambda i: (0, 0)),
          pl.BlockSpec((NC, _UBLK, BATCH), lambda i: (0, i, 0)),
          pl.BlockSpec((1, _UBLK), lambda i: (0, i)),
      ],
      out_specs=pl.BlockSpec((BATCH, _UBLK), lambda i: (0, i)),
      out_shape=jax.ShapeDtypeStruct((BATCH, UNITS), jnp.float32),
  )(eye, partials, bias.reshape(1, UNITS))


def kernel(inputs, kernel_weights, rows, cols, bias):
  pad = C_PAD - CONN
  pad_idx = jnp.arange(CONN, C_PAD, dtype=jnp.int32)
  xT = inputs.T
  w_p = jnp.pad(kernel_weights, (0, pad))
  rows_p = jnp.concatenate([rows, pad_idx % INPUT_DIM])
  cols_p = jnp.concatenate([cols, pad_idx % UNITS])
  partials = _sc_spmm(xT, w_p, rows_p, cols_p)
  return _combine(partials.reshape(NC, UNITS, BATCH), bias)
